# baseline (speedup 1.0000x reference)
import functools

import jax
import jax.numpy as jnp
from jax import lax
from jax.experimental import pallas as pl
from jax.experimental.pallas import tpu as pltpu
from jax.experimental.pallas import tpu_sc as plsc

_TN = 4096


def _mlp_body(emb_ref, W1_ref, b1_ref, W2_ref, b2_ref, p_ref, h_ref):
    step = pl.program_id(0)

    @pl.when(step == 0)
    def _prologue():
        h_ref[...] = jnp.tanh(
            jnp.dot(emb_ref[...], W1_ref[...],
                    preferred_element_type=jnp.float32) + b1_ref[...])

    p_ref[...] = jnp.dot(h_ref[...], W2_ref[...],
                         preferred_element_type=jnp.float32) + b2_ref[...]


def _project_table(emb, W1, b1, W2, b2):
    V, D = emb.shape
    H = W1.shape[1]
    N = W2.shape[1]
    return pl.pallas_call(
        _mlp_body,
        grid=(N // _TN,),
        in_specs=[
            pl.BlockSpec((V, D), lambda i: (0, 0)),
            pl.BlockSpec((D, H), lambda i: (0, 0)),
            pl.BlockSpec((1, H), lambda i: (0, 0)),
            pl.BlockSpec((D, _TN), lambda i: (0, i)),
            pl.BlockSpec((1, _TN), lambda i: (0, i)),
        ],
        out_specs=pl.BlockSpec((V, _TN), lambda i: (0, i)),
        out_shape=jax.ShapeDtypeStruct((V, N), jnp.float32),
        scratch_shapes=[pltpu.VMEM((V, H), jnp.float32)],
    )(emb, W1, b1.reshape(1, H), W2, b2.reshape(1, N))


@functools.lru_cache(maxsize=None)
def _make_sc_gather(V, N, B):
    info = plsc.get_sparse_core_info()
    NW = info.num_cores * info.num_subcores
    b_per_w = B // NW
    mesh = plsc.VectorSubcoreMesh(core_axis_name="c", subcore_axis_name="s")

    @functools.partial(
        pl.kernel,
        mesh=mesh,
        out_type=jax.ShapeDtypeStruct((B, N), jnp.float32),
        scratch_types=[
            pltpu.VMEM((b_per_w, 1), jnp.int32),
            pltpu.VMEM((1, N), jnp.float32),
            pltpu.VMEM((1, N), jnp.float32),
            pltpu.SemaphoreType.DMA,
            pltpu.SemaphoreType.DMA,
        ],
    )
    def gather(table_hbm, idx_hbm, out_hbm, idx_v, buf0, buf1, sem0, sem1):
        wid = lax.axis_index("s") * info.num_cores + lax.axis_index("c")
        base = wid * b_per_w
        pltpu.sync_copy(idx_hbm.at[pl.ds(base, b_per_w)], idx_v)
        bufs = (buf0, buf1)
        sems = (sem0, sem1)
        pltpu.async_copy(table_hbm.at[idx_v.at[0]], buf0, sem0)
        for j in range(b_per_w):
            pltpu.make_async_copy(
                table_hbm.at[idx_v.at[j]], bufs[j % 2],
                sems[j % 2]).wait()
            if j + 1 < b_per_w:
                pltpu.async_copy(
                    table_hbm.at[idx_v.at[j + 1]],
                    bufs[(j + 1) % 2], sems[(j + 1) % 2])
            pltpu.sync_copy(bufs[j % 2], out_hbm.at[pl.ds(base + j, 1)])

    return gather


def kernel(prefix, emb, W1, b1, W2, b2):
    B, L = prefix.shape
    T = B * L
    V, D = emb.shape
    N = W2.shape[1]
    p_all = _project_table(emb, W1, b1, W2, b2)
    idx = prefix.reshape(T, 1).astype(jnp.int32)
    out = _make_sc_gather(V, N, T)(p_all, idx)
    return out.reshape(B, L, N)


# speedup vs baseline: 1.6347x; 1.6347x over previous
import jax
import jax.numpy as jnp
from jax.experimental import pallas as pl
from jax.experimental.pallas import tpu as pltpu

_TN = 2048


def _body(idx_ref, emb_ref, W1_ref, b1_ref, W2_ref, b2_ref, out_ref,
          h_ref, oh_ref):
    step = pl.program_id(0)

    @pl.when(step == 0)
    def _prologue():
        h_ref[...] = jnp.tanh(
            jnp.dot(emb_ref[...], W1_ref[...],
                    preferred_element_type=jnp.float32) + b1_ref[...])
        T, V = oh_ref.shape
        iota = jax.lax.broadcasted_iota(jnp.int32, (T, V), 1)
        oh_ref[...] = (idx_ref[...] == iota).astype(jnp.float32)

    p = jnp.dot(h_ref[...], W2_ref[...],
                preferred_element_type=jnp.float32) + b2_ref[...]
    out_ref[...] = jnp.dot(oh_ref[...], p,
                           preferred_element_type=jnp.float32)


def kernel(prefix, emb, W1, b1, W2, b2):
    B, L = prefix.shape
    T = B * L
    V, D = emb.shape
    H = W1.shape[1]
    N = W2.shape[1]
    idx = prefix.reshape(T, 1).astype(jnp.int32)
    b1r = b1.reshape(1, H)
    b2r = b2.reshape(1, N)
    grid = N // _TN

    out = pl.pallas_call(
        _body,
        grid=(grid,),
        in_specs=[
            pl.BlockSpec((T, 1), lambda i: (0, 0)),
            pl.BlockSpec((V, D), lambda i: (0, 0)),
            pl.BlockSpec((D, H), lambda i: (0, 0)),
            pl.BlockSpec((1, H), lambda i: (0, 0)),
            pl.BlockSpec((D, _TN), lambda i: (0, i)),
            pl.BlockSpec((1, _TN), lambda i: (0, i)),
        ],
        out_specs=pl.BlockSpec((T, _TN), lambda i: (0, i)),
        out_shape=jax.ShapeDtypeStruct((T, N), jnp.float32),
        scratch_shapes=[
            pltpu.VMEM((V, H), jnp.float32),
            pltpu.VMEM((T, V), jnp.float32),
        ],
    )(idx, emb, W1, b1r, W2, b2r)
    return out.reshape(B, L, N)
